# Initial kernel scaffold; baseline (speedup 1.0000x reference)
import jax
import jax.numpy as jnp
from jax.experimental import pallas as pl


def kernel(rois, W1, b1, Wc, bc, Wr, br):
    raise NotImplementedError("write your pallas kernel here")


# speedup vs baseline: 1.5082x; 1.5082x over previous
import jax
import jax.numpy as jnp
from jax.experimental import pallas as pl
from jax.experimental.pallas import tpu as pltpu

_TN = 1000
_PAD_OUT = 128


def _fused_mlp_kernel(x_ref, w1_ref, b1_ref, wcr_ref, bcr_ref, out_ref):
    h = jnp.dot(x_ref[...], w1_ref[...], preferred_element_type=jnp.float32)
    h = h + b1_ref[...]
    out = jnp.dot(h, wcr_ref[...], preferred_element_type=jnp.float32)
    out_ref[...] = out + bcr_ref[...]


def kernel(rois, W1, b1, Wc, bc, Wr, br):
    x = rois[0]
    n, k = x.shape
    f = W1.shape[1]
    nc = Wc.shape[1]
    nr = Wr.shape[1]

    wcr = jnp.concatenate([Wc, Wr], axis=1)
    wcr = jnp.pad(wcr, ((0, 0), (0, _PAD_OUT - nc - nr)))
    bcr = jnp.pad(jnp.concatenate([bc, br]), (0, _PAD_OUT - nc - nr))

    grid = (n // _TN,)
    out = pl.pallas_call(
        _fused_mlp_kernel,
        grid=grid,
        in_specs=[
            pl.BlockSpec((_TN, k), lambda i: (i, 0)),
            pl.BlockSpec((k, f), lambda i: (0, 0)),
            pl.BlockSpec((1, f), lambda i: (0, 0)),
            pl.BlockSpec((f, _PAD_OUT), lambda i: (0, 0)),
            pl.BlockSpec((1, _PAD_OUT), lambda i: (0, 0)),
        ],
        out_specs=pl.BlockSpec((_TN, _PAD_OUT), lambda i: (i, 0)),
        out_shape=jax.ShapeDtypeStruct((n, _PAD_OUT), jnp.float32),
        compiler_params=pltpu.CompilerParams(
            dimension_semantics=("arbitrary",),
        ),
    )(x, W1, b1.reshape(1, f), wcr, bcr.reshape(1, _PAD_OUT))

    clss = out[:, :nc]
    reg = out[:, nc:nc + nr]
    return (reg[None, :, :], clss[None, :, :])
